# baseline (speedup 1.0000x reference)
import jax
import jax.numpy as jnp
from jax import lax
from jax.experimental import pallas as pl
from jax.experimental.pallas import tpu as pltpu
from jax.experimental.pallas import tpu_sc as plsc

VOCAB = 1000000
D = 128
B = 8192
NC = 2
NS = 16
NW = NC * NS
BPW = B // NW
CHUNKS = (32, 96, 128)
OFFS = (0, 32, 128)
NCH = len(CHUNKS)


def _body(idx_hbm, tok_hbm, pos_hbm, out_hbm, idx_v, rows_v, *sems):
    sem_i = sems[0]
    sem_p = sems[1:1 + NCH]
    sem_g = sems[1 + NCH:1 + 2 * NCH]
    sem_o = sems[1 + 2 * NCH]
    wid = lax.axis_index("s") * NC + lax.axis_index("c")
    base = wid * BPW

    def copy_pos(k):
        return pltpu.async_copy(
            pos_hbm.at[pl.ds(base + OFFS[k], CHUNKS[k]), :],
            rows_v.at[pl.ds(OFFS[k], CHUNKS[k]), :],
            sem_p[k],
        )

    def gather_add(k):
        return pltpu.async_copy(
            tok_hbm.at[idx_v.at[pl.ds(OFFS[k], CHUNKS[k])]],
            rows_v.at[pl.ds(OFFS[k], CHUNKS[k]), :],
            sem_g[k],
            add=True,
        )

    def copy_out(k):
        return pltpu.async_copy(
            rows_v.at[pl.ds(OFFS[k], CHUNKS[k]), :],
            out_hbm.at[pl.ds(base + OFFS[k], CHUNKS[k]), :],
            sem_o,
        )

    idx_c = pltpu.async_copy(idx_hbm.at[wid], idx_v, sem_i)
    pos_c = [copy_pos(0)]
    idx_c.wait()
    g_c, o_c = [], []
    for k in range(NCH):
        pos_c[k].wait()
        g_c.append(gather_add(k))
        if k + 1 < NCH:
            pos_c.append(copy_pos(k + 1))
        if k >= 1:
            g_c[k - 1].wait()
            o_c.append(copy_out(k - 1))
    g_c[NCH - 1].wait()
    o_c.append(copy_out(NCH - 1))
    for c in o_c:
        c.wait()


@jax.jit
def _embed(index, tok_table, pos_table):
    mesh = plsc.VectorSubcoreMesh(core_axis_name="c", subcore_axis_name="s")
    kfn = pl.kernel(
        _body,
        out_type=jax.ShapeDtypeStruct((B, D), jnp.float32),
        mesh=mesh,
        scratch_types=[
            pltpu.VMEM((BPW,), jnp.int32),
            pltpu.VMEM((BPW, D), jnp.float32),
        ] + [pltpu.SemaphoreType.DMA] * (2 * NCH + 2),
    )
    return kfn(index.reshape(NW, BPW), tok_table, pos_table)


def kernel(index, tok_table, pos_table):
    return _embed(index.astype(jnp.int32), tok_table, pos_table)


# speedup vs baseline: 1.0666x; 1.0666x over previous
import jax
import jax.numpy as jnp
from jax import lax
from jax.experimental import pallas as pl
from jax.experimental.pallas import tpu as pltpu
from jax.experimental.pallas import tpu_sc as plsc

VOCAB = 1000000
D = 128
B = 8192
NC = 2
NS = 16
NW = NC * NS
BPW = B // NW
CHUNKS = (32, 96, 128)
OFFS = (0, 32, 128)
NCH = len(CHUNKS)


def _body(idx_hbm, tok_hbm, pos_hbm, out_hbm, idx_v, rows_v, *sems):
    sem_i = sems[0]
    sem_p = sems[1:1 + NCH]
    sem_g = sems[1 + NCH:1 + 2 * NCH]
    sem_o = sems[1 + 2 * NCH]
    wid = lax.axis_index("s") * NC + lax.axis_index("c")
    base = wid * BPW

    def copy_pos(k):
        return pltpu.async_copy(
            pos_hbm.at[pl.ds(base + OFFS[k], CHUNKS[k]), :],
            rows_v.at[pl.ds(OFFS[k], CHUNKS[k]), :],
            sem_p[k],
        )

    def gather_add(k):
        return pltpu.async_copy(
            tok_hbm.at[idx_v.at[pl.ds(OFFS[k], CHUNKS[k])]],
            rows_v.at[pl.ds(OFFS[k], CHUNKS[k]), :],
            sem_g[k],
            add=True,
        )

    def copy_out(k):
        return pltpu.async_copy(
            rows_v.at[pl.ds(OFFS[k], CHUNKS[k]), :],
            out_hbm.at[pl.ds(base + OFFS[k], CHUNKS[k]), :],
            sem_o,
        )

    idx_c = pltpu.async_copy(idx_hbm.at[wid], idx_v, sem_i)
    pos_c = [copy_pos(k) for k in range(NCH)]
    idx_c.wait()
    g_c = []
    for k in range(NCH):
        pos_c[k].wait()
        g_c.append(gather_add(k))
    o_c = []
    for k in range(NCH):
        g_c[k].wait()
        o_c.append(copy_out(k))
    for c in o_c:
        c.wait()


@jax.jit
def _embed(index, tok_table, pos_table):
    mesh = plsc.VectorSubcoreMesh(core_axis_name="c", subcore_axis_name="s")
    kfn = pl.kernel(
        _body,
        out_type=jax.ShapeDtypeStruct((B, D), jnp.float32),
        mesh=mesh,
        scratch_types=[
            pltpu.VMEM((BPW,), jnp.int32),
            pltpu.VMEM((BPW, D), jnp.float32),
        ] + [pltpu.SemaphoreType.DMA] * (2 * NCH + 2),
    )
    return kfn(index.reshape(NW, BPW), tok_table, pos_table)


def kernel(index, tok_table, pos_table):
    return _embed(index.astype(jnp.int32), tok_table, pos_table)
